# Initial kernel scaffold; baseline (speedup 1.0000x reference)
import jax
import jax.numpy as jnp
from jax.experimental import pallas as pl


def kernel(data, x, edge, gamma, beta):
    raise NotImplementedError("write your pallas kernel here")


# speedup vs baseline: 2.5713x; 2.5713x over previous
import functools

import jax
import jax.numpy as jnp
from jax import lax
from jax.experimental import pallas as pl
from jax.experimental.pallas import tpu as pltpu
from jax.experimental.pallas import tpu_sc as plsc

N_NODES = 10000
N_EDGES = 160000
D = 256
EPS = 1e-5

LANES = 16
E_BLK = 128
N_CHUNKS = N_EDGES // E_BLK
NW = 32
CHUNKS_PER_W = -(-N_CHUNKS // NW)


def _ln_body(x_ref, g_ref, b_ref, o_ref):
    x = x_ref[...]
    mean = jnp.mean(x, axis=-1, keepdims=True)
    var = jnp.mean((x - mean) ** 2, axis=-1, keepdims=True)
    o_ref[...] = (x - mean) * lax.rsqrt(var + EPS) * g_ref[...] + b_ref[...]


def _layernorm(x, gamma, beta):
    blk = 2000
    return pl.pallas_call(
        _ln_body,
        grid=(N_NODES // blk,),
        in_specs=[
            pl.BlockSpec((blk, D), lambda i: (i, 0)),
            pl.BlockSpec((D,), lambda i: (0,)),
            pl.BlockSpec((D,), lambda i: (0,)),
        ],
        out_specs=pl.BlockSpec((blk, D), lambda i: (i, 0)),
        out_shape=jax.ShapeDtypeStruct((N_NODES, D), jnp.float32),
    )(x, gamma, beta)


_MESH = plsc.VectorSubcoreMesh(core_axis_name="c", subcore_axis_name="s")


@functools.partial(
    pl.kernel,
    out_type=jax.ShapeDtypeStruct((N_EDGES, D), jnp.float32),
    mesh=_MESH,
    scratch_types=[
        pltpu.VMEM((E_BLK,), jnp.int32),
        pltpu.VMEM((E_BLK,), jnp.int32),
        pltpu.VMEM((E_BLK, D), jnp.float32),
        pltpu.VMEM((E_BLK, D), jnp.float32),
        pltpu.SemaphoreType.DMA,
    ],
)
def _gather_mul(xn_hbm, src_hbm, dst_hbm, out_hbm,
                si_v, di_v, a_v, b_v, sem):
    wid = lax.axis_index("s") * 2 + lax.axis_index("c")

    def chunk_body(t, _):
        c = wid + t * NW

        @pl.when(c < N_CHUNKS)
        def _():
            base = c * E_BLK
            pltpu.sync_copy(src_hbm.at[pl.ds(base, E_BLK)], si_v)
            pltpu.sync_copy(dst_hbm.at[pl.ds(base, E_BLK)], di_v)
            ca = pltpu.async_copy(xn_hbm.at[si_v], a_v, sem)
            cb = pltpu.async_copy(xn_hbm.at[di_v], b_v, sem)
            ca.wait()
            cb.wait()

            def mul_row(e, _):
                for j in range(D // LANES):
                    s = pl.ds(j * LANES, LANES)
                    a_v[e, s] = a_v[e, s] * b_v[e, s]
                return 0

            lax.fori_loop(0, E_BLK, mul_row, 0)
            pltpu.sync_copy(a_v, out_hbm.at[pl.ds(base, E_BLK)])

        return 0

    lax.fori_loop(0, CHUNKS_PER_W, chunk_body, 0)


def kernel(data, x, edge, gamma, beta):
    xn = _layernorm(x, gamma, beta)
    src = edge[0]
    dst = edge[1]
    return _gather_mul(xn, src, dst)


# speedup vs baseline: 4.3003x; 1.6724x over previous
import functools

import jax
import jax.numpy as jnp
from jax import lax
from jax.experimental import pallas as pl
from jax.experimental.pallas import tpu as pltpu
from jax.experimental.pallas import tpu_sc as plsc

N_NODES = 10000
N_EDGES = 160000
D = 256
EPS = 1e-5

LANES = 16
E_BLK = 80
N_CHUNKS = N_EDGES // E_BLK
NW = 32
MAX_STEPS = -(-N_CHUNKS // NW)


def _ln_body(x_ref, g_ref, b_ref, o_ref):
    x = x_ref[...]
    mean = jnp.mean(x, axis=-1, keepdims=True)
    var = jnp.mean((x - mean) ** 2, axis=-1, keepdims=True)
    o_ref[...] = (x - mean) * lax.rsqrt(var + EPS) * g_ref[...] + b_ref[...]


def _layernorm(x, gamma, beta):
    blk = 2000
    return pl.pallas_call(
        _ln_body,
        grid=(N_NODES // blk,),
        in_specs=[
            pl.BlockSpec((blk, D), lambda i: (i, 0)),
            pl.BlockSpec((D,), lambda i: (0,)),
            pl.BlockSpec((D,), lambda i: (0,)),
        ],
        out_specs=pl.BlockSpec((blk, D), lambda i: (i, 0)),
        out_shape=jax.ShapeDtypeStruct((N_NODES, D), jnp.float32),
    )(x, gamma, beta)


_MESH = plsc.VectorSubcoreMesh(core_axis_name="c", subcore_axis_name="s")


@functools.partial(
    pl.kernel,
    out_type=jax.ShapeDtypeStruct((N_EDGES, D), jnp.float32),
    mesh=_MESH,
    scratch_types=[
        pltpu.VMEM((E_BLK,), jnp.int32),
        pltpu.VMEM((E_BLK,), jnp.int32),
        pltpu.VMEM((E_BLK, D), jnp.float32),
        pltpu.VMEM((E_BLK, D), jnp.float32),
        pltpu.VMEM((E_BLK,), jnp.int32),
        pltpu.VMEM((E_BLK,), jnp.int32),
        pltpu.VMEM((E_BLK, D), jnp.float32),
        pltpu.VMEM((E_BLK, D), jnp.float32),
        pltpu.SemaphoreType.DMA,
        pltpu.SemaphoreType.DMA,
        pltpu.SemaphoreType.DMA,
        pltpu.SemaphoreType.DMA,
        pltpu.SemaphoreType.DMA,
        pltpu.SemaphoreType.DMA,
    ],
)
def _gather_mul(xn_hbm, src_hbm, dst_hbm, out_hbm,
                si0, di0, a0, b0, si1, di1, a1, b1,
                gi0, gi1, g0, g1, w0, w1):
    wid = lax.axis_index("s") * 2 + lax.axis_index("c")
    SI, DI, A, B = (si0, si1), (di0, di1), (a0, a1), (b0, b1)
    GI, G, W = (gi0, gi1), (g0, g1), (w0, w1)

    def cbase(s):
        return (wid + s * NW) * E_BLK

    def valid(s):
        return (wid + s * NW) < N_CHUNKS

    def fire_idx(s, k):
        pltpu.async_copy(src_hbm.at[pl.ds(cbase(s), E_BLK)], SI[k], GI[k])
        pltpu.async_copy(dst_hbm.at[pl.ds(cbase(s), E_BLK)], DI[k], GI[k])

    def wait_idx(k):
        pltpu.make_async_copy(src_hbm.at[pl.ds(0, E_BLK)], SI[k], GI[k]).wait()
        pltpu.make_async_copy(dst_hbm.at[pl.ds(0, E_BLK)], DI[k], GI[k]).wait()

    def fire_gathers(k):
        pltpu.async_copy(xn_hbm.at[SI[k]], A[k], G[k])
        pltpu.async_copy(xn_hbm.at[DI[k]], B[k], G[k])

    def wait_gathers(k):
        pltpu.make_async_copy(xn_hbm.at[pl.ds(0, E_BLK)], A[k], G[k]).wait()
        pltpu.make_async_copy(xn_hbm.at[pl.ds(0, E_BLK)], B[k], G[k]).wait()

    def fire_wb(s, k):
        pltpu.async_copy(A[k], out_hbm.at[pl.ds(cbase(s), E_BLK)], W[k])

    def wait_wb(k):
        pltpu.make_async_copy(A[k], out_hbm.at[pl.ds(0, E_BLK)], W[k]).wait()

    def multiply(k):
        ak, bk = A[k], B[k]

        def row(e, _):
            for j in range(D // LANES):
                sl = pl.ds(j * LANES, LANES)
                ak[e, sl] = ak[e, sl] * bk[e, sl]
            return 0

        lax.fori_loop(0, E_BLK, row, 0)

    fire_idx(0, 0)
    fire_idx(1, 1)
    wait_idx(0)
    fire_gathers(0)

    def step(s, k):
        kn = 1 - k

        @pl.when(jnp.logical_and(s >= 1, valid(s - 1)))
        def _():
            wait_wb(kn)

        @pl.when(valid(s + 1))
        def _():
            wait_idx(kn)
            fire_gathers(kn)

        @pl.when(valid(s))
        def _():
            wait_gathers(k)

        @pl.when(valid(s + 2))
        def _():
            fire_idx(s + 2, k)

        @pl.when(valid(s))
        def _():
            multiply(k)
            fire_wb(s, k)

    def pair(i, _):
        step(2 * i, 0)
        step(2 * i + 1, 1)
        return 0

    lax.fori_loop(0, (MAX_STEPS + 2) // 2, pair, 0)


def kernel(data, x, edge, gamma, beta):
    xn = _layernorm(x, gamma, beta)
    src = edge[0]
    dst = edge[1]
    return _gather_mul(xn, src, dst)
